# Initial kernel scaffold; baseline (speedup 1.0000x reference)
import jax
import jax.numpy as jnp
from jax.experimental import pallas as pl


def kernel(pred0, pred1, pred2, targets):
    raise NotImplementedError("write your pallas kernel here")


# speedup vs baseline: 2.0896x; 2.0896x over previous
import functools

import jax
import jax.numpy as jnp
import numpy as np
from jax import lax
from jax.experimental import pallas as pl
from jax.experimental.pallas import tpu as pltpu
from jax.experimental.pallas import tpu_sc as plsc

_ANCHORS = [[1.25, 1.625, 2.0, 3.75, 4.125, 2.875],
            [1.875, 3.8125, 3.875, 2.8125, 3.6875, 7.4375],
            [3.625, 2.8125, 4.875, 6.1875, 11.65625, 10.1875]]
_HW = [(64, 64), (32, 32), (16, 16)]
_NT = 200
_TP = 256
_NE = 5 * _TP
_NW = 32
_BPW = _NE // _NW
_GRID = 24
_ROWS = (5440, 1360, 680)
_CHUNK = (64, 16, 8)
_M = (196608.0, 49152.0, 12288.0)
_TROWS = (130560, 32640, 8160)


def _softplus(x):
    return jnp.maximum(x, 0.0) + jnp.log1p(jnp.exp(-jnp.abs(x)))


def _sigmoid(x):
    return 1.0 / (1.0 + jnp.exp(-x))


def _atan_pos(u):
    inv = u > 1.0
    z = jnp.where(inv, 1.0 / u, u)
    z2 = z * z
    p = z * (0.99997726 + z2 * (-0.33262347 + z2 * (0.19354346
         + z2 * (-0.11643287 + z2 * (0.05265332 + z2 * (-0.01172120))))))
    return jnp.where(inv, float(np.pi / 2) - p, p)


def _prep_body(t_ref, c0_ref, c1_ref, c2_ref, pd_ref):
    t = t_ref[...]
    oi = lax.broadcasted_iota(jnp.int32, (5, 1), 0)
    ox = 0.5 * ((oi == 1).astype(jnp.float32) - (oi == 3).astype(jnp.float32))
    oy = 0.5 * ((oi == 2).astype(jnp.float32) - (oi == 4).astype(jnp.float32))
    crefs = (c0_ref, c1_ref, c2_ref)
    for l in range(3):
        h, w = _HW[l]
        anch = np.asarray(_ANCHORS[l], np.float32).reshape(3, 2)
        tb = t[0:1, :]
        tc = t[1:2, :]
        tx = t[2:3, :] * float(w)
        ty = t[3:4, :] * float(h)
        tw = t[4:5, :] * float(w)
        th = t[5:6, :] * float(h)
        ksum = jnp.zeros_like(tx)
        for a in range(3):
            rw = tw / float(anch[a, 0])
            rh = th / float(anch[a, 1])
            ok = jnp.maximum(jnp.maximum(rw, 1.0 / rw),
                             jnp.maximum(rh, 1.0 / rh)) < 4.0
            ksum = ksum + ok.astype(jnp.float32)
        jmx = ((jnp.mod(tx, 1.0) < 0.5) & (tx > 1.0)).astype(jnp.float32)
        jmy = ((jnp.mod(ty, 1.0) < 0.5) & (ty > 1.0)).astype(jnp.float32)
        gxx = float(w) - tx
        gxy = float(h) - ty
        lmx = ((jnp.mod(gxx, 1.0) < 0.5) & (gxx > 1.0)).astype(jnp.float32)
        lmy = ((jnp.mod(gxy, 1.0) < 0.5) & (gxy > 1.0)).astype(jnp.float32)
        offmask = jnp.concatenate(
            [jnp.ones_like(jmx), jmx, jmy, lmx, lmy], axis=0)
        cnt = offmask * ksum
        gix = (tx - ox).astype(jnp.int32)
        giy = (ty - oy).astype(jnp.int32)
        gi = jnp.clip(gix, 0, w - 1)
        gj = jnp.clip(giy, 0, h - 1)
        bint = tb.astype(jnp.int32)
        cell = ((bint * 3 + 1) * h + gj) * w + gi
        tbx = tx - gi.astype(jnp.float32)
        tby = ty - gj.astype(jnp.float32)
        cs = cell * 85
        r0 = lax.shift_right_logical(cs, 7)
        r1 = jnp.minimum(r0 + 1, _TROWS[l] - 1)
        sh = jnp.bitwise_and(cs, 127)
        crefs[l][...] = jnp.concatenate(
            [r0.reshape(1, _NE), r1.reshape(1, _NE)], axis=0)
        five = (5, _TP)
        rows = [tbx, tby,
                jnp.broadcast_to(tw, five), jnp.broadcast_to(th, five),
                cnt, jnp.broadcast_to(tc, five),
                sh.astype(jnp.float32), jnp.zeros(five, jnp.float32)]
        pd_ref[l, :, :] = jnp.concatenate(
            [r.reshape(1, _NE) for r in rows], axis=0)


_SC_NC = 2


@functools.lru_cache(maxsize=1)
def _make_sc_gather():
    mesh = plsc.VectorSubcoreMesh(core_axis_name="c", subcore_axis_name="s")

    @functools.partial(
        pl.kernel,
        mesh=mesh,
        out_type=tuple(jax.ShapeDtypeStruct((2 * _NE, 128), jnp.float32)
                       for _ in range(3)),
        scratch_types=[
            pltpu.VMEM((2 * _BPW,), jnp.int32),
            pltpu.VMEM((2 * _BPW, 128), jnp.float32),
            pltpu.SemaphoreType.DMA,
        ],
    )
    def _sc_gather(t0, t1, t2, c0, c1, c2, o0, o1, o2, idx_v, rows_v, sem):
        wid = lax.axis_index("s") * _SC_NC + lax.axis_index("c")
        base = wid * 2 * _BPW
        for tbl, cid, out in ((t0, c0, o0), (t1, c1, o1), (t2, c2, o2)):
            pltpu.sync_copy(cid.at[wid], idx_v)
            pltpu.async_copy(tbl.at[idx_v], rows_v, sem).wait()
            pltpu.sync_copy(rows_v, out.at[pl.ds(base, 2 * _BPW)])

    return _sc_gather


def _entry_losses(g_ref, pd, l, acc):
    ax0 = float(_ANCHORS[l][0])
    ay0 = float(_ANCHORS[l][1])
    x = g_ref[...]
    it = lax.broadcasted_iota(jnp.int32, (_NE, 256), 1)
    sh = pd[:, 6:7].astype(jnp.int32)
    rel = it - sh
    sp = _softplus(x)
    spc = jnp.sum(jnp.where((rel >= 5) & (rel < 85), sp, 0.0),
                  axis=1, keepdims=True)
    clsi = pd[:, 5:6].astype(jnp.int32) + 5
    csel = jnp.sum(jnp.where(rel == clsi, x, 0.0), axis=1, keepdims=True)
    col = [jnp.sum(jnp.where(rel == k, x, 0.0), axis=1, keepdims=True)
           for k in range(5)]
    px = _sigmoid(col[0]) * 2.0 - 0.5
    py = _sigmoid(col[1]) * 2.0 - 0.5
    pw = (_sigmoid(col[2]) * 2.0) ** 2 * ax0
    ph = (_sigmoid(col[3]) * 2.0) ** 2 * ay0
    xobj = col[4]
    tbx = pd[:, 0:1]
    tby = pd[:, 1:2]
    tw = pd[:, 2:3]
    th = pd[:, 3:4]
    cnt = pd[:, 4:5]
    eps = 1e-7
    b1x1, b1x2 = px - pw * 0.5, px + pw * 0.5
    b1y1, b1y2 = py - ph * 0.5, py + ph * 0.5
    b2x1, b2x2 = tbx - tw * 0.5, tbx + tw * 0.5
    b2y1, b2y2 = tby - th * 0.5, tby + th * 0.5
    inter = (jnp.clip(jnp.minimum(b1x2, b2x2) - jnp.maximum(b1x1, b2x1), 0, None)
             * jnp.clip(jnp.minimum(b1y2, b2y2) - jnp.maximum(b1y1, b2y1), 0, None))
    w1, h1 = b1x2 - b1x1, b1y2 - b1y1 + eps
    w2, h2 = b2x2 - b2x1, b2y2 - b2y1 + eps
    union = w1 * h1 + w2 * h2 - inter + eps
    iou = inter / union
    cw = jnp.maximum(b1x2, b2x2) - jnp.minimum(b1x1, b2x1)
    ch = jnp.maximum(b1y2, b2y2) - jnp.minimum(b1y1, b2y1)
    c2 = cw * cw + ch * ch + eps
    rho2 = ((b2x1 + b2x2 - b1x1 - b1x2) ** 2
            + (b2y1 + b2y2 - b1y1 - b1y2) ** 2) * 0.25
    v = (4.0 / (np.pi ** 2)) * (_atan_pos(w2 / h2) - _atan_pos(w1 / h1)) ** 2
    alpha = v / (v - iou + (1.0 + eps))
    ciou = iou - (rho2 / c2 + v * alpha)
    acc[3 + 4 * l] = jnp.sum(cnt * (1.0 - ciou))
    acc[4 + 4 * l] = jnp.sum(cnt)
    acc[5 + 4 * l] = jnp.sum(cnt * (spc - csel))
    acc[6 + 4 * l] = jnp.sum(
        jnp.where(cnt > 0.0, 1.0, 0.0) * xobj * jnp.clip(ciou, 0.0, None))


def _main_body(a0, a1, a2, m0, m1, m2, s0, s1, s2,
               g0, g1, g2, pd_ref, out_ref, acc):
    i = pl.program_id(0)
    parts = []
    for a_ref, m_ref, s_ref in ((a0, m0, s0), (a1, m1, s1), (a2, m2, s2)):
        xm = a_ref[...] * m_ref[...]
        comp = jnp.dot(s_ref[...], xm, preferred_element_type=jnp.float32)
        parts.append(jnp.sum(_softplus(comp)))

    @pl.when(i == 0)
    def _first():
        acc[0] = parts[0]
        acc[1] = parts[1]
        acc[2] = parts[2]
        pd = pd_ref[...]
        _entry_losses(g0, pd[0], 0, acc)
        _entry_losses(g1, pd[1], 1, acc)
        _entry_losses(g2, pd[2], 2, acc)

    @pl.when(i > 0)
    def _accum():
        acc[0] = acc[0] + parts[0]
        acc[1] = acc[1] + parts[1]

    @pl.when(jnp.logical_and(i > 0, i % 2 == 0))
    def _accum2():
        acc[2] = acc[2] + parts[2]

    @pl.when(i == _GRID - 1)
    def _fin():
        lbox = jnp.float32(0.0)
        lobj = jnp.float32(0.0)
        lcls = jnp.float32(0.0)
        for l in range(3):
            den = jnp.maximum(acc[4 + 4 * l], 1.0)
            lbox = lbox + acc[3 + 4 * l] / den
            lcls = lcls + acc[5 + 4 * l] / (den * 80.0)
            lobj = lobj + (acc[l] - acc[6 + 4 * l]) / _M[l]
        lbox = lbox * 0.05
        lcls = lcls * 0.5
        loss = lbox + lobj + lcls
        out_ref[0] = lbox
        out_ref[1] = lobj
        out_ref[2] = lcls
        out_ref[3] = loss
        out_ref[4] = 0.0
        out_ref[5] = 0.0
        out_ref[6] = 0.0
        out_ref[7] = 0.0


def _build_consts():
    base = (jnp.arange(85 * 128, dtype=jnp.int32) % 85 == 4)
    base = base.reshape(85, 128).astype(jnp.float32)
    ms = tuple(jnp.tile(base, (c, 1)) for c in _CHUNK)
    ss = tuple(
        (jnp.arange(c, dtype=jnp.int32)[:, None]
         == jnp.arange(r, dtype=jnp.int32)[None, :] // 85).astype(jnp.float32)
        for c, r in zip(_CHUNK, _ROWS))
    return ms, ss


def kernel(pred0, pred1, pred2, targets):
    preds = (pred0, pred1, pred2)
    tpad = jnp.pad(targets.astype(jnp.float32).T, ((0, 1), (0, _TP - _NT)))

    c0, c1, c2, pd = pl.pallas_call(
        _prep_body,
        out_shape=(
            jax.ShapeDtypeStruct((2, _NE), jnp.int32),
            jax.ShapeDtypeStruct((2, _NE), jnp.int32),
            jax.ShapeDtypeStruct((2, _NE), jnp.int32),
            jax.ShapeDtypeStruct((3, 8, _NE), jnp.float32),
        ),
    )(tpad)

    ileave = lambda c: (c.reshape(2, _NW, _BPW).transpose(1, 2, 0)
                        .reshape(_NW, 2 * _BPW))
    c0, c1, c2 = ileave(c0), ileave(c1), ileave(c2)
    pd = pd.transpose(0, 2, 1)

    (m0, m1, m2), (s0, s1, s2) = _build_consts()
    flats = tuple(p.reshape(-1, 128) for p in preds)

    g0, g1, g2 = _make_sc_gather()(flats[0], flats[1], flats[2], c0, c1, c2)
    g0, g1, g2 = (g.reshape(_NE, 256) for g in (g0, g1, g2))

    def rowspec(l):
        if l == 2:
            return pl.BlockSpec((_ROWS[2], 128), lambda i: (i // 2, 0))
        return pl.BlockSpec((_ROWS[l], 128), lambda i, l=l: (i, 0))

    const2 = lambda shape: pl.BlockSpec(shape, lambda i: (0, 0))
    out = pl.pallas_call(
        _main_body,
        grid=(_GRID,),
        in_specs=[
            rowspec(0), rowspec(1), rowspec(2),
            const2((_ROWS[0], 128)), const2((_ROWS[1], 128)),
            const2((_ROWS[2], 128)),
            const2((_CHUNK[0], _ROWS[0])), const2((_CHUNK[1], _ROWS[1])),
            const2((_CHUNK[2], _ROWS[2])),
            const2((_NE, 256)), const2((_NE, 256)), const2((_NE, 256)),
            pl.BlockSpec((3, _NE, 8), lambda i: (0, 0, 0)),
        ],
        out_specs=pl.BlockSpec(memory_space=pltpu.SMEM),
        out_shape=jax.ShapeDtypeStruct((8,), jnp.float32),
        scratch_shapes=[pltpu.SMEM((16,), jnp.float32)],
    )(flats[0], flats[1], flats[2], m0, m1, m2, s0, s1, s2, g0, g1, g2, pd)

    return out[3:4], out[0:4]


# speedup vs baseline: 2.1834x; 1.0449x over previous
import functools

import jax
import jax.numpy as jnp
import numpy as np
from jax import lax
from jax.experimental import pallas as pl
from jax.experimental.pallas import tpu as pltpu
from jax.experimental.pallas import tpu_sc as plsc

_ANCHORS = [[1.25, 1.625, 2.0, 3.75, 4.125, 2.875],
            [1.875, 3.8125, 3.875, 2.8125, 3.6875, 7.4375],
            [3.625, 2.8125, 4.875, 6.1875, 11.65625, 10.1875]]
_HW = [(64, 64), (32, 32), (16, 16)]
_NT = 200
_TP = 256
_NE = 5 * _TP
_NW = 32
_BPW = _NE // _NW
_GRID = 24
_ROWS = (5440, 1360, 680)
_CHUNK = (64, 16, 8)
_M = (196608.0, 49152.0, 12288.0)
_TROWS = (130560, 32640, 8160)


def _softplus(x):
    return jnp.maximum(x, 0.0) + jnp.log1p(jnp.exp(-jnp.abs(x)))


def _sigmoid(x):
    return 1.0 / (1.0 + jnp.exp(-x))


def _atan_pos(u):
    inv = u > 1.0
    z = jnp.where(inv, 1.0 / u, u)
    z2 = z * z
    p = z * (0.99997726 + z2 * (-0.33262347 + z2 * (0.19354346
         + z2 * (-0.11643287 + z2 * (0.05265332 + z2 * (-0.01172120))))))
    return jnp.where(inv, float(np.pi / 2) - p, p)


def _prep_body(t_ref, c0_ref, c1_ref, c2_ref, pd_ref):
    t = t_ref[...]
    oi = lax.broadcasted_iota(jnp.int32, (5, 1), 0)
    ox = 0.5 * ((oi == 1).astype(jnp.float32) - (oi == 3).astype(jnp.float32))
    oy = 0.5 * ((oi == 2).astype(jnp.float32) - (oi == 4).astype(jnp.float32))
    crefs = (c0_ref, c1_ref, c2_ref)
    for l in range(3):
        h, w = _HW[l]
        anch = np.asarray(_ANCHORS[l], np.float32).reshape(3, 2)
        tb = t[0:1, :]
        tc = t[1:2, :]
        tx = t[2:3, :] * float(w)
        ty = t[3:4, :] * float(h)
        tw = t[4:5, :] * float(w)
        th = t[5:6, :] * float(h)
        ksum = jnp.zeros_like(tx)
        for a in range(3):
            rw = tw / float(anch[a, 0])
            rh = th / float(anch[a, 1])
            ok = jnp.maximum(jnp.maximum(rw, 1.0 / rw),
                             jnp.maximum(rh, 1.0 / rh)) < 4.0
            ksum = ksum + ok.astype(jnp.float32)
        jmx = ((jnp.mod(tx, 1.0) < 0.5) & (tx > 1.0)).astype(jnp.float32)
        jmy = ((jnp.mod(ty, 1.0) < 0.5) & (ty > 1.0)).astype(jnp.float32)
        gxx = float(w) - tx
        gxy = float(h) - ty
        lmx = ((jnp.mod(gxx, 1.0) < 0.5) & (gxx > 1.0)).astype(jnp.float32)
        lmy = ((jnp.mod(gxy, 1.0) < 0.5) & (gxy > 1.0)).astype(jnp.float32)
        offmask = jnp.concatenate(
            [jnp.ones_like(jmx), jmx, jmy, lmx, lmy], axis=0)
        cnt = offmask * ksum
        gix = (tx - ox).astype(jnp.int32)
        giy = (ty - oy).astype(jnp.int32)
        gi = jnp.clip(gix, 0, w - 1)
        gj = jnp.clip(giy, 0, h - 1)
        bint = tb.astype(jnp.int32)
        cell = ((bint * 3 + 1) * h + gj) * w + gi
        tbx = tx - gi.astype(jnp.float32)
        tby = ty - gj.astype(jnp.float32)
        cs = cell * 85
        r0 = lax.shift_right_logical(cs, 7)
        r1 = jnp.minimum(r0 + 1, _TROWS[l] - 1)
        sh = jnp.bitwise_and(cs, 127)
        crefs[l][...] = jnp.concatenate(
            [r0.reshape(1, _NE), r1.reshape(1, _NE)], axis=0)
        five = (5, _TP)
        rows = [tbx, tby,
                jnp.broadcast_to(tw, five), jnp.broadcast_to(th, five),
                cnt, jnp.broadcast_to(tc, five),
                sh.astype(jnp.float32), jnp.zeros(five, jnp.float32)]
        pd_ref[l, :, :] = jnp.concatenate(
            [r.reshape(1, _NE) for r in rows], axis=0)


_SC_NC = 2


@functools.lru_cache(maxsize=1)
def _make_sc_gather():
    mesh = plsc.VectorSubcoreMesh(core_axis_name="c", subcore_axis_name="s")

    @functools.partial(
        pl.kernel,
        mesh=mesh,
        out_type=tuple(jax.ShapeDtypeStruct((2 * _NE, 128), jnp.float32)
                       for _ in range(3)),
        scratch_types=[
            pltpu.VMEM((2 * _BPW,), jnp.int32),
            pltpu.VMEM((2 * _BPW,), jnp.int32),
            pltpu.VMEM((2 * _BPW,), jnp.int32),
            pltpu.VMEM((2 * _BPW, 128), jnp.float32),
            pltpu.VMEM((2 * _BPW, 128), jnp.float32),
            pltpu.VMEM((2 * _BPW, 128), jnp.float32),
            pltpu.SemaphoreType.DMA,
        ],
    )
    def _sc_gather(t0, t1, t2, c0, c1, c2, o0, o1, o2,
                   i0, i1, i2, r0, r1, r2, sem):
        wid = lax.axis_index("s") * _SC_NC + lax.axis_index("c")
        base = wid * 2 * _BPW
        tri = ((t0, c0, o0, i0, r0), (t1, c1, o1, i1, r1),
               (t2, c2, o2, i2, r2))
        copies = []
        for tbl, cid, out, idx_v, rows_v in tri:
            pltpu.sync_copy(cid.at[wid], idx_v)
            copies.append(pltpu.async_copy(tbl.at[idx_v], rows_v, sem))
        for (tbl, cid, out, idx_v, rows_v), cp in zip(tri, copies):
            cp.wait()
            pltpu.sync_copy(rows_v, out.at[pl.ds(base, 2 * _BPW)])

    return _sc_gather


def _entry_losses(g_ref, pd, l, acc):
    ax0 = float(_ANCHORS[l][0])
    ay0 = float(_ANCHORS[l][1])
    x = g_ref[...]
    it = lax.broadcasted_iota(jnp.int32, (_NE, 256), 1)
    ones = (lax.broadcasted_iota(jnp.int32, (256, 8), 0) >= 0
            ).astype(jnp.float32)[:, 0:1]
    lsum = lambda y: jnp.dot(y, ones, preferred_element_type=jnp.float32)
    sh = pd[:, 6:7].astype(jnp.int32)
    rel = it - sh
    sp = _softplus(x)
    spc = lsum(jnp.where((rel >= 5) & (rel < 85), sp, 0.0))
    clsi = pd[:, 5:6].astype(jnp.int32) + 5
    csel = lsum(jnp.where(rel == clsi, x, 0.0))
    col = [lsum(jnp.where(rel == k, x, 0.0)) for k in range(5)]
    px = _sigmoid(col[0]) * 2.0 - 0.5
    py = _sigmoid(col[1]) * 2.0 - 0.5
    pw = (_sigmoid(col[2]) * 2.0) ** 2 * ax0
    ph = (_sigmoid(col[3]) * 2.0) ** 2 * ay0
    xobj = col[4]
    tbx = pd[:, 0:1]
    tby = pd[:, 1:2]
    tw = pd[:, 2:3]
    th = pd[:, 3:4]
    cnt = pd[:, 4:5]
    eps = 1e-7
    b1x1, b1x2 = px - pw * 0.5, px + pw * 0.5
    b1y1, b1y2 = py - ph * 0.5, py + ph * 0.5
    b2x1, b2x2 = tbx - tw * 0.5, tbx + tw * 0.5
    b2y1, b2y2 = tby - th * 0.5, tby + th * 0.5
    inter = (jnp.clip(jnp.minimum(b1x2, b2x2) - jnp.maximum(b1x1, b2x1), 0, None)
             * jnp.clip(jnp.minimum(b1y2, b2y2) - jnp.maximum(b1y1, b2y1), 0, None))
    w1, h1 = b1x2 - b1x1, b1y2 - b1y1 + eps
    w2, h2 = b2x2 - b2x1, b2y2 - b2y1 + eps
    union = w1 * h1 + w2 * h2 - inter + eps
    iou = inter / union
    cw = jnp.maximum(b1x2, b2x2) - jnp.minimum(b1x1, b2x1)
    ch = jnp.maximum(b1y2, b2y2) - jnp.minimum(b1y1, b2y1)
    c2 = cw * cw + ch * ch + eps
    rho2 = ((b2x1 + b2x2 - b1x1 - b1x2) ** 2
            + (b2y1 + b2y2 - b1y1 - b1y2) ** 2) * 0.25
    v = (4.0 / (np.pi ** 2)) * (_atan_pos(w2 / h2) - _atan_pos(w1 / h1)) ** 2
    alpha = v / (v - iou + (1.0 + eps))
    ciou = iou - (rho2 / c2 + v * alpha)
    acc[3 + 4 * l] = jnp.sum(cnt * (1.0 - ciou))
    acc[4 + 4 * l] = jnp.sum(cnt)
    acc[5 + 4 * l] = jnp.sum(cnt * (spc - csel))
    acc[6 + 4 * l] = jnp.sum(
        jnp.where(cnt > 0.0, 1.0, 0.0) * xobj * jnp.clip(ciou, 0.0, None))


def _main_body(a0, a1, a2, m0, m1, m2, s0, s1, s2,
               g0, g1, g2, pd_ref, out_ref, acc):
    i = pl.program_id(0)
    parts = []
    for a_ref, m_ref, s_ref in ((a0, m0, s0), (a1, m1, s1), (a2, m2, s2)):
        xm = a_ref[...] * m_ref[...]
        comp = jnp.dot(s_ref[...], xm, preferred_element_type=jnp.float32)
        parts.append(jnp.sum(_softplus(comp)))

    @pl.when(i == 0)
    def _first():
        acc[0] = parts[0]
        acc[1] = parts[1]
        acc[2] = parts[2]
        pd = pd_ref[...]
        _entry_losses(g0, pd[0], 0, acc)
        _entry_losses(g1, pd[1], 1, acc)
        _entry_losses(g2, pd[2], 2, acc)

    @pl.when(i > 0)
    def _accum():
        acc[0] = acc[0] + parts[0]
        acc[1] = acc[1] + parts[1]

    @pl.when(jnp.logical_and(i > 0, i % 2 == 0))
    def _accum2():
        acc[2] = acc[2] + parts[2]

    @pl.when(i == _GRID - 1)
    def _fin():
        lbox = jnp.float32(0.0)
        lobj = jnp.float32(0.0)
        lcls = jnp.float32(0.0)
        for l in range(3):
            den = jnp.maximum(acc[4 + 4 * l], 1.0)
            lbox = lbox + acc[3 + 4 * l] / den
            lcls = lcls + acc[5 + 4 * l] / (den * 80.0)
            lobj = lobj + (acc[l] - acc[6 + 4 * l]) / _M[l]
        lbox = lbox * 0.05
        lcls = lcls * 0.5
        loss = lbox + lobj + lcls
        out_ref[0] = lbox
        out_ref[1] = lobj
        out_ref[2] = lcls
        out_ref[3] = loss
        out_ref[4] = 0.0
        out_ref[5] = 0.0
        out_ref[6] = 0.0
        out_ref[7] = 0.0


def _build_consts():
    base = (jnp.arange(85 * 128, dtype=jnp.int32) % 85 == 4)
    base = base.reshape(85, 128).astype(jnp.float32)
    ms = tuple(jnp.tile(base, (c, 1)) for c in _CHUNK)
    ss = tuple(
        (jnp.arange(c, dtype=jnp.int32)[:, None]
         == jnp.arange(r, dtype=jnp.int32)[None, :] // 85).astype(jnp.float32)
        for c, r in zip(_CHUNK, _ROWS))
    return ms, ss


def kernel(pred0, pred1, pred2, targets):
    preds = (pred0, pred1, pred2)
    tpad = jnp.pad(targets.astype(jnp.float32).T, ((0, 1), (0, _TP - _NT)))

    c0, c1, c2, pd = pl.pallas_call(
        _prep_body,
        out_shape=(
            jax.ShapeDtypeStruct((2, _NE), jnp.int32),
            jax.ShapeDtypeStruct((2, _NE), jnp.int32),
            jax.ShapeDtypeStruct((2, _NE), jnp.int32),
            jax.ShapeDtypeStruct((3, 8, _NE), jnp.float32),
        ),
    )(tpad)

    ileave = lambda c: (c.reshape(2, _NW, _BPW).transpose(1, 2, 0)
                        .reshape(_NW, 2 * _BPW))
    c0, c1, c2 = ileave(c0), ileave(c1), ileave(c2)
    pd = pd.transpose(0, 2, 1)

    (m0, m1, m2), (s0, s1, s2) = _build_consts()
    flats = tuple(p.reshape(-1, 128) for p in preds)

    g0, g1, g2 = _make_sc_gather()(flats[0], flats[1], flats[2], c0, c1, c2)
    g0, g1, g2 = (g.reshape(_NE, 256) for g in (g0, g1, g2))

    def rowspec(l):
        if l == 2:
            return pl.BlockSpec((_ROWS[2], 128), lambda i: (i // 2, 0))
        return pl.BlockSpec((_ROWS[l], 128), lambda i, l=l: (i, 0))

    const2 = lambda shape: pl.BlockSpec(shape, lambda i: (0, 0))
    out = pl.pallas_call(
        _main_body,
        grid=(_GRID,),
        in_specs=[
            rowspec(0), rowspec(1), rowspec(2),
            const2((_ROWS[0], 128)), const2((_ROWS[1], 128)),
            const2((_ROWS[2], 128)),
            const2((_CHUNK[0], _ROWS[0])), const2((_CHUNK[1], _ROWS[1])),
            const2((_CHUNK[2], _ROWS[2])),
            const2((_NE, 256)), const2((_NE, 256)), const2((_NE, 256)),
            pl.BlockSpec((3, _NE, 8), lambda i: (0, 0, 0)),
        ],
        out_specs=pl.BlockSpec(memory_space=pltpu.SMEM),
        out_shape=jax.ShapeDtypeStruct((8,), jnp.float32),
        scratch_shapes=[pltpu.SMEM((16,), jnp.float32)],
    )(flats[0], flats[1], flats[2], m0, m1, m2, s0, s1, s2, g0, g1, g2, pd)

    return out[3:4], out[0:4]
